# baseline (speedup 1.0000x reference)
import functools

import jax
import jax.numpy as jnp
from jax import lax
from jax.experimental import pallas as pl
from jax.experimental.pallas import tpu as pltpu
from jax.experimental.pallas import tpu_sc as plsc

MAXLEN = 200
EMBED = 64
BATCH = 4096

NC, NS, LANES = 2, 16, 16
NW = NC * NS
BT = BATCH * MAXLEN
B_PER_W = BT // NW
SEQ_PER_W = B_PER_W // MAXLEN
NBUF = 4
SPLIT = (104, 96)


def _body(x_hbm, tok_hbm, pos_hbm, out_hbm, idx_v, pos_v, rows_v, sem_g, sem_s):
    wid = lax.axis_index("c") * NS + lax.axis_index("s")
    base = wid * B_PER_W

    pltpu.sync_copy(x_hbm.at[pl.ds(base, B_PER_W)], idx_v)
    pltpu.sync_copy(pos_hbm, pos_v)

    def gather_start(s, b):
        off = s * MAXLEN
        pltpu.async_copy(
            tok_hbm.at[idx_v.at[pl.ds(off, SPLIT[0])]],
            rows_v.at[b, pl.ds(0, SPLIT[0])], sem_g.at[b])
        pltpu.async_copy(
            tok_hbm.at[idx_v.at[pl.ds(off + SPLIT[0], SPLIT[1])]],
            rows_v.at[b, pl.ds(SPLIT[0], SPLIT[1])], sem_g.at[b])

    def gather_wait(s, b):
        off = s * MAXLEN
        pltpu.make_async_copy(
            tok_hbm.at[idx_v.at[pl.ds(off, SPLIT[0])]],
            rows_v.at[b, pl.ds(0, SPLIT[0])], sem_g.at[b]).wait()
        pltpu.make_async_copy(
            tok_hbm.at[idx_v.at[pl.ds(off + SPLIT[0], SPLIT[1])]],
            rows_v.at[b, pl.ds(SPLIT[0], SPLIT[1])], sem_g.at[b]).wait()

    def scatter_start(s, b):
        pltpu.async_copy(
            rows_v.at[b], out_hbm.at[pl.ds(base + s * MAXLEN, MAXLEN)],
            sem_s.at[b])

    def scatter_wait(s, b):
        pltpu.make_async_copy(
            rows_v.at[b], out_hbm.at[pl.ds(base + s * MAXLEN, MAXLEN)],
            sem_s.at[b]).wait()

    def add_pos(b):
        @pl.loop(0, MAXLEN, unroll=4)
        def _(r):
            for d in range(EMBED // LANES):
                plsc.addupdate(
                    rows_v.at[b, r, pl.ds(d * LANES, LANES)],
                    pos_v[r, pl.ds(d * LANES, LANES)])

    for b in range(2):
        gather_start(b, b)

    @pl.loop(0, SEQ_PER_W, step=NBUF)
    def _(s0):
        for b in range(NBUF):
            s = s0 + b
            bn = (b + 2) % NBUF

            @pl.when(s >= 2)
            def _():
                scatter_wait(s - 2, bn)

            @pl.when(s + 2 < SEQ_PER_W)
            def _():
                gather_start(s + 2, bn)

            gather_wait(s, b)
            add_pos(b)
            scatter_start(s, b)

    scatter_wait(SEQ_PER_W - 2, (SEQ_PER_W - 2) % NBUF)
    scatter_wait(SEQ_PER_W - 1, (SEQ_PER_W - 1) % NBUF)


@functools.partial(jax.jit, static_argnames=())
def _run(x_flat, token_table, pos_table):
    mesh = plsc.VectorSubcoreMesh(core_axis_name="c", subcore_axis_name="s")
    return pl.kernel(
        _body,
        out_type=jax.ShapeDtypeStruct((BT, EMBED), jnp.float32),
        mesh=mesh,
        compiler_params=pltpu.CompilerParams(use_tc_tiling_on_sc=False),
        scratch_types=[
            pltpu.VMEM((B_PER_W,), jnp.int32),
            pltpu.VMEM((MAXLEN, EMBED), jnp.float32),
            pltpu.VMEM((NBUF, MAXLEN, EMBED), jnp.float32),
            pltpu.SemaphoreType.DMA((NBUF,)),
            pltpu.SemaphoreType.DMA((NBUF,)),
        ],
    )(x_flat, token_table, pos_table)


def kernel(x, token_table, pos_table):
    x_flat = x.reshape(BT).astype(jnp.int32)
    out = _run(x_flat, token_table, pos_table)
    return out.reshape(BATCH, MAXLEN, EMBED)


# speedup vs baseline: 1.4208x; 1.4208x over previous
import functools

import jax
import jax.numpy as jnp
from jax import lax
from jax.experimental import pallas as pl
from jax.experimental.pallas import tpu as pltpu
from jax.experimental.pallas import tpu_sc as plsc

MAXLEN = 200
VOCAB = 1000000
EMBED = 64
BATCH = 4096

NC, NS, LANES = 2, 16, 16
NW = NC * NS
BT = BATCH * MAXLEN
B_PER_W = BT // NW
SEQ_PER_W = B_PER_W // MAXLEN
NBUF = 4
SPLIT = (104, 96)


def _body(x_hbm, tok_hbm, pos_hbm, out_hbm, idx_v, pos_v, rows_v, sem_g, sem_s):
    wid = lax.axis_index("c") * NS + lax.axis_index("s")
    base = wid * B_PER_W

    pltpu.sync_copy(x_hbm.at[pl.ds(base, B_PER_W)], idx_v)
    pltpu.sync_copy(pos_hbm, pos_v)

    def gather_start(s, b):
        off = s * MAXLEN
        pltpu.async_copy(
            tok_hbm.at[idx_v.at[pl.ds(off, SPLIT[0])]],
            rows_v.at[b, pl.ds(0, SPLIT[0])], sem_g.at[b])
        pltpu.async_copy(
            tok_hbm.at[idx_v.at[pl.ds(off + SPLIT[0], SPLIT[1])]],
            rows_v.at[b, pl.ds(SPLIT[0], SPLIT[1])], sem_g.at[b])

    def gather_wait(s, b):
        off = s * MAXLEN
        pltpu.make_async_copy(
            tok_hbm.at[idx_v.at[pl.ds(off, SPLIT[0])]],
            rows_v.at[b, pl.ds(0, SPLIT[0])], sem_g.at[b]).wait()
        pltpu.make_async_copy(
            tok_hbm.at[idx_v.at[pl.ds(off + SPLIT[0], SPLIT[1])]],
            rows_v.at[b, pl.ds(SPLIT[0], SPLIT[1])], sem_g.at[b]).wait()

    def scatter_start(s, b):
        pltpu.async_copy(
            rows_v.at[b],
            out_hbm.at[wid * SEQ_PER_W + s, :, pl.ds(0, EMBED)], sem_s.at[b])

    def scatter_wait(s, b):
        pltpu.make_async_copy(
            rows_v.at[b],
            out_hbm.at[wid * SEQ_PER_W + s, :, pl.ds(0, EMBED)],
            sem_s.at[b]).wait()

    def add_pos(b):
        @pl.loop(0, MAXLEN, unroll=4)
        def _(r):
            for d in range(EMBED // LANES):
                plsc.addupdate(
                    rows_v.at[b, r, pl.ds(d * LANES, LANES)],
                    pos_v[r, pl.ds(d * LANES, LANES)])

    for b in range(2):
        gather_start(b, b)

    @pl.loop(0, SEQ_PER_W, step=NBUF)
    def _(s0):
        for b in range(NBUF):
            s = s0 + b
            bn = (b + 2) % NBUF

            @pl.when(s >= 2)
            def _():
                scatter_wait(s - 2, bn)

            @pl.when(s + 2 < SEQ_PER_W)
            def _():
                gather_start(s + 2, bn)

            gather_wait(s, b)
            add_pos(b)
            scatter_start(s, b)

    scatter_wait(SEQ_PER_W - 2, (SEQ_PER_W - 2) % NBUF)
    scatter_wait(SEQ_PER_W - 1, (SEQ_PER_W - 1) % NBUF)


@jax.jit
def _run(x2, tok2, pos_table):
    mesh = plsc.VectorSubcoreMesh(core_axis_name="c", subcore_axis_name="s")
    return pl.kernel(
        _body,
        out_type=jax.ShapeDtypeStruct((NW * SEQ_PER_W, MAXLEN, 128),
                                      jnp.float32),
        mesh=mesh,
        compiler_params=pltpu.CompilerParams(use_tc_tiling_on_sc=False),
        scratch_types=[
            pltpu.VMEM((B_PER_W,), jnp.int32),
            pltpu.VMEM((MAXLEN, EMBED), jnp.float32),
            pltpu.VMEM((NBUF, MAXLEN, EMBED), jnp.float32),
            pltpu.SemaphoreType.DMA((NBUF,)),
            pltpu.SemaphoreType.DMA((NBUF,)),
        ],
    )(x2, tok2, pos_table)


def kernel(x, token_table, pos_table):
    tok2 = jnp.pad(token_table, ((0, 0), (0, 128 - EMBED))).reshape(
        2 * VOCAB, EMBED)
    x2 = x.reshape(BT).astype(jnp.int32) * 2
    return _run(x2, tok2, pos_table)[:, :, :EMBED]
